# Initial kernel scaffold; baseline (speedup 1.0000x reference)
import jax
import jax.numpy as jnp
from jax.experimental import pallas as pl


def kernel(feats0, feats1, adj_rows, adj_cols, adj_vals, warm_idx, W0, b0, g0, be0, W1, b1, g1, be1, fuse_w, Wf, bf):
    raise NotImplementedError("write your pallas kernel here")


# speedup vs baseline: 10.0440x; 10.0440x over previous
import jax
import jax.numpy as jnp
from jax import lax
from jax.experimental import pallas as pl
from jax.experimental.pallas import tpu as pltpu
from jax.experimental.pallas import tpu_sc as plsc

N_USERS = 50000
N_ITEMS = 50000
N_WARM = 40000
D0 = 256
D1 = 128
HID = 128
EMB = 32

R = 2000
NB = N_ITEMS // R

NC = 2
NS = 16
NW = NC * NS
KE = 512
GRP = KE // 128
RCH = 400
NRCH = N_USERS // RCH
NREP = (NRCH + NS - 1) // NS
NWCH = (N_WARM + 127) // 128
WPAD = NWCH * 128


def _l2n(x, eps=1e-12):
    n = jnp.sqrt(jnp.sum(x * x, axis=-1, keepdims=True))
    return x / jnp.maximum(n, eps)


def _k1_body(f0_ref, f1_ref, w0_ref, w1_ref, x0_ref, x1_ref, st_ref):
    i = pl.program_id(0)
    f0 = _l2n(f0_ref[...])
    x0 = jnp.dot(f0, w0_ref[...], preferred_element_type=jnp.float32,
                 precision=lax.Precision.HIGHEST)
    x0_ref[...] = x0
    f1 = _l2n(f1_ref[...])
    x1 = jnp.dot(f1, w1_ref[...], preferred_element_type=jnp.float32,
                 precision=lax.Precision.HIGHEST)
    x1_ref[...] = x1

    @pl.when(i == 0)
    def _():
        st_ref[...] = jnp.zeros_like(st_ref)

    st_ref[0:1, :] += jnp.sum(x0, axis=0, keepdims=True)
    st_ref[1:2, :] += jnp.sum(x0 * x0, axis=0, keepdims=True)
    st_ref[2:3, :] += jnp.sum(x1, axis=0, keepdims=True)
    st_ref[3:4, :] += jnp.sum(x1 * x1, axis=0, keepdims=True)


_k1 = pl.pallas_call(
    _k1_body,
    grid=(NB,),
    in_specs=[
        pl.BlockSpec((R, D0), lambda i: (i, 0)),
        pl.BlockSpec((R, D1), lambda i: (i, 0)),
        pl.BlockSpec((D0, HID), lambda i: (0, 0)),
        pl.BlockSpec((D1, HID), lambda i: (0, 0)),
    ],
    out_specs=[
        pl.BlockSpec((R, HID), lambda i: (i, 0)),
        pl.BlockSpec((R, HID), lambda i: (i, 0)),
        pl.BlockSpec((8, HID), lambda i: (0, 0)),
    ],
    out_shape=[
        jax.ShapeDtypeStruct((N_ITEMS, HID), jnp.float32),
        jax.ShapeDtypeStruct((N_ITEMS, HID), jnp.float32),
        jax.ShapeDtypeStruct((8, HID), jnp.float32),
    ],
)


def _k2_body(x0_ref, x1_ref, p_ref, wf_ref, nf_ref):
    h = (jax.nn.relu(x0_ref[...] * p_ref[0:1, :] + p_ref[1:2, :])
         + jax.nn.relu(x1_ref[...] * p_ref[2:3, :] + p_ref[3:4, :]))
    o = jnp.dot(h, wf_ref[...], preferred_element_type=jnp.float32,
                precision=lax.Precision.HIGHEST) + p_ref[4:5, 0:EMB]
    nf_ref[...] = _l2n(o)


_k2 = pl.pallas_call(
    _k2_body,
    grid=(NB,),
    in_specs=[
        pl.BlockSpec((R, HID), lambda i: (i, 0)),
        pl.BlockSpec((R, HID), lambda i: (i, 0)),
        pl.BlockSpec((8, HID), lambda i: (0, 0)),
        pl.BlockSpec((HID, EMB), lambda i: (0, 0)),
    ],
    out_specs=pl.BlockSpec((R, EMB), lambda i: (i, 0)),
    out_shape=jax.ShapeDtypeStruct((N_ITEMS, EMB), jnp.float32),
)


def _sca_body(nf_hbm, warm_hbm, item_hbm, idx_v, emb_v, sem):
    c = lax.axis_index("c")
    s = lax.axis_index("s")
    wid = c * NS + s
    nrep = (NWCH + NW - 1) // NW
    for rep in range(nrep):
        k = rep * NW + wid

        @pl.when(k < NWCH)
        def _():
            pltpu.sync_copy(warm_hbm.at[pl.ds(k, 1)], idx_v)
            pltpu.async_copy(nf_hbm.at[idx_v.at[0]], emb_v, sem).wait()
            pltpu.sync_copy(
                emb_v, item_hbm.at[pl.ds(pl.multiple_of(k * 128, 8), 128)])


_sca = pl.kernel(
    _sca_body,
    out_type=jax.ShapeDtypeStruct((WPAD, EMB), jnp.float32),
    mesh=plsc.VectorSubcoreMesh(core_axis_name="c", subcore_axis_name="s",
                                num_cores=NC, num_subcores=NS),
    compiler_params=pltpu.CompilerParams(needs_layout_passes=False,
                                         use_tc_tiling_on_sc=False),
    scratch_types=[
        pltpu.VMEM((1, 128), jnp.int32),
        pltpu.VMEM((128, EMB), jnp.float32),
        pltpu.SemaphoreType.DMA,
    ],
)


def _sc_body(item_hbm, cols_hbm, rows_hbm, vals_hbm, out_hbm,
             part_sh, cols_v, rows_v, vals_v, gath_v, sem_g, sem_s):
    c = lax.axis_index("c")
    s = lax.axis_index("s")
    wid = c * NS + s
    n_iter = vals_hbm.shape[0] // (NW * KE)
    tile_edges = n_iter * KE

    def _z(j, cry):
        gath_v[j, pl.ds(0, 16)] = jnp.zeros((16,), jnp.float32)
        gath_v[j, pl.ds(16, 16)] = jnp.zeros((16,), jnp.float32)
        return cry

    lax.fori_loop(0, RCH, _z, 0)
    for rep in range(NREP):
        k = rep * NS + s

        @pl.when(k < NRCH)
        def _():
            pltpu.sync_copy(
                gath_v.at[pl.ds(0, RCH)],
                part_sh.at[pl.ds(pl.multiple_of(k * RCH, 8), RCH)])

    plsc.subcore_barrier()

    def _iter(i, cry):
        base = pl.multiple_of((wid * tile_edges + i * KE), KE)
        gbase = pl.multiple_of((wid * tile_edges + i * KE) // 128, GRP)
        pltpu.sync_copy(cols_hbm.at[pl.ds(gbase, GRP)], cols_v)
        pltpu.sync_copy(rows_hbm.at[pl.ds(gbase, GRP)], rows_v)
        pltpu.sync_copy(vals_hbm.at[pl.ds(base, KE)], vals_v)

        descs = []
        for g in range(GRP):
            descs.append(pltpu.async_copy(
                item_hbm.at[cols_v.at[g]],
                gath_v.at[pl.ds(g * 128, 128)], sem_g))
        for d in descs:
            d.wait()

        def _sc16(j, cry2):
            e0 = j * 16
            for mm in range(16):
                b = plsc.load_gather(
                    vals_v, [jnp.full((16,), e0 + mm, jnp.int32)])
                gath_v[e0 + mm, pl.ds(0, 16)] = \
                    gath_v[e0 + mm, pl.ds(0, 16)] * b
                gath_v[e0 + mm, pl.ds(16, 16)] = \
                    gath_v[e0 + mm, pl.ds(16, 16)] * b
            return cry2

        lax.fori_loop(0, KE // 16, _sc16, 0)

        descs2 = []
        for g in range(GRP):
            descs2.append(pltpu.async_copy(
                gath_v.at[pl.ds(g * 128, 128)],
                part_sh.at[rows_v.at[g]], sem_s, add=True))
        for d in descs2:
            d.wait()
        return cry

    lax.fori_loop(0, n_iter, _iter, 0)

    plsc.subcore_barrier()
    for rep in range(NREP):
        k = rep * NS + s

        @pl.when(k < NRCH)
        def _():
            pltpu.sync_copy(
                part_sh.at[pl.ds(pl.multiple_of(k * RCH, 8), RCH)],
                out_hbm.at[pl.ds(pl.multiple_of(c * N_USERS + k * RCH, 8),
                                 RCH)])


_scb = pl.kernel(
    _sc_body,
    out_type=jax.ShapeDtypeStruct((NC * N_USERS, EMB), jnp.float32),
    mesh=plsc.VectorSubcoreMesh(core_axis_name="c", subcore_axis_name="s",
                                num_cores=NC, num_subcores=NS),
    compiler_params=pltpu.CompilerParams(needs_layout_passes=False,
                                         use_tc_tiling_on_sc=False),
    scratch_types=[
        pltpu.VMEM_SHARED((N_USERS, EMB), jnp.float32),
        pltpu.VMEM((GRP, 128), jnp.int32),
        pltpu.VMEM((GRP, 128), jnp.int32),
        pltpu.VMEM((KE,), jnp.float32),
        pltpu.VMEM((KE, EMB), jnp.float32),
        pltpu.SemaphoreType.DMA,
        pltpu.SemaphoreType.DMA,
    ],
)


def _k4_body(p0_ref, p1_ref, u_ref):
    u_ref[...] = _l2n(p0_ref[...] + p1_ref[...])


_k4 = pl.pallas_call(
    _k4_body,
    grid=(NB,),
    in_specs=[
        pl.BlockSpec((R, EMB), lambda i: (i, 0)),
        pl.BlockSpec((R, EMB), lambda i: (NB + i, 0)),
    ],
    out_specs=pl.BlockSpec((R, EMB), lambda i: (i, 0)),
    out_shape=jax.ShapeDtypeStruct((N_USERS, EMB), jnp.float32),
)


def kernel(feats0, feats1, adj_rows, adj_cols, adj_vals, warm_idx,
           W0, b0, g0, be0, W1, b1, g1, be1, fuse_w, Wf, bf):
    x0, x1, st = _k1(feats0, feats1, W0, W1)

    n = jnp.float32(N_ITEMS)
    w = jax.nn.softmax(fuse_w)
    mu0 = st[0] / n
    var0 = st[1] / n - mu0 * mu0
    sr0 = g0 * lax.rsqrt(var0 + 1e-5)
    mu1 = st[2] / n
    var1 = st[3] / n - mu1 * mu1
    sr1 = g1 * lax.rsqrt(var1 + 1e-5)
    p = jnp.zeros((8, HID), jnp.float32)
    p = p.at[0].set(w[0] * sr0)
    p = p.at[1].set(w[0] * (be0 - mu0 * sr0))
    p = p.at[2].set(w[1] * sr1)
    p = p.at[3].set(w[1] * (be1 - mu1 * sr1))
    p = p.at[4, 0:EMB].set(bf)

    nf = _k2(x0, x1, p, Wf)

    e = adj_rows.shape[0]
    chunk = NW * KE
    epad = ((e + chunk - 1) // chunk) * chunk
    pad = epad - e
    rows_p = jnp.pad(adj_rows, (0, pad)).reshape(epad // 128, 128)
    cols_p = jnp.pad(adj_cols, (0, pad)).reshape(epad // 128, 128)
    vals_p = jnp.pad(adj_vals, (0, pad))

    warm_p = jnp.pad(warm_idx, (0, WPAD - N_WARM)).reshape(NWCH, 128)
    item = _sca(nf, warm_p)
    partials = _scb(item, cols_p, rows_p, vals_p)
    user = _k4(partials, partials)
    return (user, nf)


# speedup vs baseline: 10.2770x; 1.0232x over previous
import jax
import jax.numpy as jnp
from jax import lax
from jax.experimental import pallas as pl
from jax.experimental.pallas import tpu as pltpu
from jax.experimental.pallas import tpu_sc as plsc

N_USERS = 50000
N_ITEMS = 50000
N_WARM = 40000
D0 = 256
D1 = 128
HID = 128
EMB = 32

R = 2000
NB = N_ITEMS // R

NC = 2
NS = 16
NW = NC * NS
KE = 256
GRP = KE // 128
RCH = 200
NRCH = N_USERS // RCH
NREP = (NRCH + NS - 1) // NS
NWCH = (N_WARM + 127) // 128
WPAD = NWCH * 128


def _l2n(x, eps=1e-12):
    n = jnp.sqrt(jnp.sum(x * x, axis=-1, keepdims=True))
    return x / jnp.maximum(n, eps)


def _k1_body(f0_ref, f1_ref, w0_ref, w1_ref, x0_ref, x1_ref, st_ref):
    i = pl.program_id(0)
    f0 = _l2n(f0_ref[...])
    x0 = jnp.dot(f0, w0_ref[...], preferred_element_type=jnp.float32,
                 precision=lax.Precision.HIGHEST)
    x0_ref[...] = x0
    f1 = _l2n(f1_ref[...])
    x1 = jnp.dot(f1, w1_ref[...], preferred_element_type=jnp.float32,
                 precision=lax.Precision.HIGHEST)
    x1_ref[...] = x1

    @pl.when(i == 0)
    def _():
        st_ref[...] = jnp.zeros_like(st_ref)

    st_ref[0:1, :] += jnp.sum(x0, axis=0, keepdims=True)
    st_ref[1:2, :] += jnp.sum(x0 * x0, axis=0, keepdims=True)
    st_ref[2:3, :] += jnp.sum(x1, axis=0, keepdims=True)
    st_ref[3:4, :] += jnp.sum(x1 * x1, axis=0, keepdims=True)


_k1 = pl.pallas_call(
    _k1_body,
    grid=(NB,),
    in_specs=[
        pl.BlockSpec((R, D0), lambda i: (i, 0)),
        pl.BlockSpec((R, D1), lambda i: (i, 0)),
        pl.BlockSpec((D0, HID), lambda i: (0, 0)),
        pl.BlockSpec((D1, HID), lambda i: (0, 0)),
    ],
    out_specs=[
        pl.BlockSpec((R, HID), lambda i: (i, 0)),
        pl.BlockSpec((R, HID), lambda i: (i, 0)),
        pl.BlockSpec((8, HID), lambda i: (0, 0)),
    ],
    out_shape=[
        jax.ShapeDtypeStruct((N_ITEMS, HID), jnp.float32),
        jax.ShapeDtypeStruct((N_ITEMS, HID), jnp.float32),
        jax.ShapeDtypeStruct((8, HID), jnp.float32),
    ],
)


def _k2_body(x0_ref, x1_ref, p_ref, wf_ref, nf_ref):
    h = (jax.nn.relu(x0_ref[...] * p_ref[0:1, :] + p_ref[1:2, :])
         + jax.nn.relu(x1_ref[...] * p_ref[2:3, :] + p_ref[3:4, :]))
    o = jnp.dot(h, wf_ref[...], preferred_element_type=jnp.float32,
                precision=lax.Precision.HIGHEST) + p_ref[4:5, 0:EMB]
    nf_ref[...] = _l2n(o)


_k2 = pl.pallas_call(
    _k2_body,
    grid=(NB,),
    in_specs=[
        pl.BlockSpec((R, HID), lambda i: (i, 0)),
        pl.BlockSpec((R, HID), lambda i: (i, 0)),
        pl.BlockSpec((8, HID), lambda i: (0, 0)),
        pl.BlockSpec((HID, EMB), lambda i: (0, 0)),
    ],
    out_specs=pl.BlockSpec((R, EMB), lambda i: (i, 0)),
    out_shape=jax.ShapeDtypeStruct((N_ITEMS, EMB), jnp.float32),
)


def _sca_body(nf_hbm, warm_hbm, item_hbm, idx_v, emb_v, sem):
    c = lax.axis_index("c")
    s = lax.axis_index("s")
    wid = c * NS + s
    nrep = (NWCH + NW - 1) // NW
    for rep in range(nrep):
        k = rep * NW + wid

        @pl.when(k < NWCH)
        def _():
            pltpu.sync_copy(warm_hbm.at[pl.ds(k, 1)], idx_v)
            pltpu.async_copy(nf_hbm.at[idx_v.at[0]], emb_v, sem).wait()
            pltpu.sync_copy(
                emb_v, item_hbm.at[pl.ds(pl.multiple_of(k * 128, 8), 128)])


_sca = pl.kernel(
    _sca_body,
    out_type=jax.ShapeDtypeStruct((WPAD, EMB), jnp.float32),
    mesh=plsc.VectorSubcoreMesh(core_axis_name="c", subcore_axis_name="s",
                                num_cores=NC, num_subcores=NS),
    compiler_params=pltpu.CompilerParams(needs_layout_passes=False,
                                         use_tc_tiling_on_sc=False),
    scratch_types=[
        pltpu.VMEM((1, 128), jnp.int32),
        pltpu.VMEM((128, EMB), jnp.float32),
        pltpu.SemaphoreType.DMA,
    ],
)


def _sc_body(item_hbm, cols_hbm, rows_hbm, vals_hbm, out_hbm,
             part_sh, cols_a, rows_a, vals_a, gath_a,
             cols_b, rows_b, vals_b, gath_b, sem_ga, sem_gb, sem_s):
    c = lax.axis_index("c")
    s = lax.axis_index("s")
    wid = c * NS + s
    n_iter = vals_hbm.shape[0] // (NW * KE)
    tile_edges = n_iter * KE

    def _z(j, cry):
        gath_a[j, pl.ds(0, 16)] = jnp.zeros((16,), jnp.float32)
        gath_a[j, pl.ds(16, 16)] = jnp.zeros((16,), jnp.float32)
        return cry

    lax.fori_loop(0, RCH, _z, 0)
    for rep in range(NREP):
        k = rep * NS + s

        @pl.when(k < NRCH)
        def _():
            pltpu.sync_copy(
                gath_a.at[pl.ds(0, RCH)],
                part_sh.at[pl.ds(pl.multiple_of(k * RCH, 8), RCH)])

    plsc.subcore_barrier()

    def _stage(i, cols_x, rows_x, vals_x):
        base = pl.multiple_of(wid * tile_edges + i * KE, KE)
        gbase = pl.multiple_of((wid * tile_edges + i * KE) // 128, GRP)
        pltpu.sync_copy(cols_hbm.at[pl.ds(gbase, GRP)], cols_x)
        pltpu.sync_copy(rows_hbm.at[pl.ds(gbase, GRP)], rows_x)
        pltpu.sync_copy(vals_hbm.at[pl.ds(base, KE)], vals_x)

    def _fire_gather(cols_x, gath_x, sem_x):
        for g in range(GRP):
            pltpu.async_copy(item_hbm.at[cols_x.at[g]],
                             gath_x.at[pl.ds(g * 128, 128)], sem_x)

    def _drain_gather(cols_x, gath_x, sem_x):
        for g in range(GRP):
            pltpu.make_async_copy(item_hbm.at[cols_x.at[g]],
                                  gath_x.at[pl.ds(g * 128, 128)],
                                  sem_x).wait()

    def _scale(vals_x, gath_x):
        def _sc16(j, cry2):
            e0 = j * 16
            for mm in range(16):
                b = plsc.load_gather(
                    vals_x, [jnp.full((16,), e0 + mm, jnp.int32)])
                gath_x[e0 + mm, pl.ds(0, 16)] = \
                    gath_x[e0 + mm, pl.ds(0, 16)] * b
                gath_x[e0 + mm, pl.ds(16, 16)] = \
                    gath_x[e0 + mm, pl.ds(16, 16)] * b
            return cry2

        lax.fori_loop(0, KE // 16, _sc16, 0)

    def _scatter(rows_x, gath_x):
        descs = []
        for g in range(GRP):
            descs.append(pltpu.async_copy(
                gath_x.at[pl.ds(g * 128, 128)],
                part_sh.at[rows_x.at[g]], sem_s, add=True))
        for d in descs:
            d.wait()

    _stage(0, cols_a, rows_a, vals_a)
    _fire_gather(cols_a, gath_a, sem_ga)

    def _pair(j, cry):
        a = 2 * j
        _stage(a + 1, cols_b, rows_b, vals_b)
        _fire_gather(cols_b, gath_b, sem_gb)
        _drain_gather(cols_a, gath_a, sem_ga)
        _scale(vals_a, gath_a)
        _scatter(rows_a, gath_a)

        @pl.when(a + 2 < n_iter)
        def _():
            _stage(a + 2, cols_a, rows_a, vals_a)
            _fire_gather(cols_a, gath_a, sem_ga)

        _drain_gather(cols_b, gath_b, sem_gb)
        _scale(vals_b, gath_b)
        _scatter(rows_b, gath_b)
        return cry

    lax.fori_loop(0, n_iter // 2, _pair, 0)

    plsc.subcore_barrier()
    for rep in range(NREP):
        k = rep * NS + s

        @pl.when(k < NRCH)
        def _():
            pltpu.sync_copy(
                part_sh.at[pl.ds(pl.multiple_of(k * RCH, 8), RCH)],
                out_hbm.at[pl.ds(pl.multiple_of(c * N_USERS + k * RCH, 8),
                                 RCH)])


_scb = pl.kernel(
    _sc_body,
    out_type=jax.ShapeDtypeStruct((NC * N_USERS, EMB), jnp.float32),
    mesh=plsc.VectorSubcoreMesh(core_axis_name="c", subcore_axis_name="s",
                                num_cores=NC, num_subcores=NS),
    compiler_params=pltpu.CompilerParams(needs_layout_passes=False,
                                         use_tc_tiling_on_sc=False),
    scratch_types=[
        pltpu.VMEM_SHARED((N_USERS, EMB), jnp.float32),
        pltpu.VMEM((GRP, 128), jnp.int32),
        pltpu.VMEM((GRP, 128), jnp.int32),
        pltpu.VMEM((KE,), jnp.float32),
        pltpu.VMEM((KE, EMB), jnp.float32),
        pltpu.VMEM((GRP, 128), jnp.int32),
        pltpu.VMEM((GRP, 128), jnp.int32),
        pltpu.VMEM((KE,), jnp.float32),
        pltpu.VMEM((KE, EMB), jnp.float32),
        pltpu.SemaphoreType.DMA,
        pltpu.SemaphoreType.DMA,
        pltpu.SemaphoreType.DMA,
    ],
)


def _k4_body(p0_ref, p1_ref, u_ref):
    u_ref[...] = _l2n(p0_ref[...] + p1_ref[...])


_k4 = pl.pallas_call(
    _k4_body,
    grid=(NB,),
    in_specs=[
        pl.BlockSpec((R, EMB), lambda i: (i, 0)),
        pl.BlockSpec((R, EMB), lambda i: (NB + i, 0)),
    ],
    out_specs=pl.BlockSpec((R, EMB), lambda i: (i, 0)),
    out_shape=jax.ShapeDtypeStruct((N_USERS, EMB), jnp.float32),
)


def kernel(feats0, feats1, adj_rows, adj_cols, adj_vals, warm_idx,
           W0, b0, g0, be0, W1, b1, g1, be1, fuse_w, Wf, bf):
    x0, x1, st = _k1(feats0, feats1, W0, W1)

    n = jnp.float32(N_ITEMS)
    w = jax.nn.softmax(fuse_w)
    mu0 = st[0] / n
    var0 = st[1] / n - mu0 * mu0
    sr0 = g0 * lax.rsqrt(var0 + 1e-5)
    mu1 = st[2] / n
    var1 = st[3] / n - mu1 * mu1
    sr1 = g1 * lax.rsqrt(var1 + 1e-5)
    p = jnp.zeros((8, HID), jnp.float32)
    p = p.at[0].set(w[0] * sr0)
    p = p.at[1].set(w[0] * (be0 - mu0 * sr0))
    p = p.at[2].set(w[1] * sr1)
    p = p.at[3].set(w[1] * (be1 - mu1 * sr1))
    p = p.at[4, 0:EMB].set(bf)

    nf = _k2(x0, x1, p, Wf)

    e = adj_rows.shape[0]
    chunk = 2 * NW * KE
    epad = ((e + chunk - 1) // chunk) * chunk
    pad = epad - e
    rows_p = jnp.pad(adj_rows, (0, pad)).reshape(epad // 128, 128)
    cols_p = jnp.pad(adj_cols, (0, pad)).reshape(epad // 128, 128)
    vals_p = jnp.pad(adj_vals, (0, pad))

    warm_p = jnp.pad(warm_idx, (0, WPAD - N_WARM)).reshape(NWCH, 128)
    item = _sca(nf, warm_p)
    partials = _scb(item, cols_p, rows_p, vals_p)
    user = _k4(partials, partials)
    return (user, nf)
